# Initial kernel scaffold; baseline (speedup 1.0000x reference)
import jax
import jax.numpy as jnp
from jax.experimental import pallas as pl


def kernel(mha_out, Wg, bg, Wn, bn, topk):
    raise NotImplementedError("write your pallas kernel here")


# speedup vs baseline: 1.6762x; 1.6762x over previous
import functools

import jax
import jax.numpy as jnp
from jax import lax
from jax.experimental import pallas as pl
from jax.experimental.pallas import tpu as pltpu

_TOKENS = 32768
_EMB = 768
_E = 8
_BLK = 2048


def _router_body(x_ref, w_ref, b_ref, sn_ref, out_ref, idx_ref):
    x = x_ref[...]
    w = w_ref[...]
    b = b_ref[...]
    acc = jnp.dot(x, w, preferred_element_type=jnp.float32) + b
    logits = acc[:, :_E]
    nlog = acc[:, _E:]
    softplus = jnp.maximum(nlog, 0.0) + jnp.log1p(jnp.exp(-jnp.abs(nlog)))
    noisy = logits + sn_ref[...] * softplus

    ii = lax.broadcasted_iota(jnp.int32, noisy.shape, 1)
    m1 = jnp.max(noisy, axis=1, keepdims=True)
    i1 = jnp.min(jnp.where(noisy == m1, ii, _E), axis=1, keepdims=True)
    rest = jnp.where(ii == i1, -jnp.inf, noisy)
    m2 = jnp.max(rest, axis=1, keepdims=True)
    i2 = jnp.min(jnp.where(rest == m2, ii, _E), axis=1, keepdims=True)

    sel = (ii == i1) | (ii == i2)
    e = jnp.where(sel, jnp.exp(noisy - m1), 0.0)
    out_ref[...] = e / jnp.sum(e, axis=1, keepdims=True)
    idx_ref[...] = jnp.concatenate([i1, i2], axis=1)


def kernel(mha_out, Wg, bg, Wn, bn, topk):
    del topk
    w = jnp.concatenate([Wg, Wn], axis=0).T
    b = jnp.concatenate([bg, bn])[None, :]
    stdnorm = jax.random.normal(jax.random.key(42), (_TOKENS, _E), jnp.float32)

    grid = (_TOKENS // _BLK,)
    out, idx = pl.pallas_call(
        _router_body,
        grid=grid,
        in_specs=[
            pl.BlockSpec((_BLK, _EMB), lambda i: (i, 0)),
            pl.BlockSpec((_EMB, 2 * _E), lambda i: (0, 0)),
            pl.BlockSpec((1, 2 * _E), lambda i: (0, 0)),
            pl.BlockSpec((_BLK, _E), lambda i: (i, 0)),
        ],
        out_specs=[
            pl.BlockSpec((_BLK, _E), lambda i: (i, 0)),
            pl.BlockSpec((_BLK, 2), lambda i: (i, 0)),
        ],
        out_shape=[
            jax.ShapeDtypeStruct((_TOKENS, _E), jnp.float32),
            jax.ShapeDtypeStruct((_TOKENS, 2), jnp.int32),
        ],
    )(mha_out, w, b, stdnorm)
    return (out, idx)
